# baseline (speedup 1.0000x reference)
import jax
import jax.numpy as jnp
from jax.experimental import pallas as pl
from jax.experimental.pallas import tpu as pltpu

_TM = 256


def _moe_dense_body(x_ref, lg_ref, w_ref, o_ref):
    logits = lg_ref[...]
    num_experts = w_ref.shape[0]
    iota = jax.lax.broadcasted_iota(jnp.int32, logits.shape, 1)

    i1 = jnp.argmax(logits, axis=1, keepdims=True)
    m1 = jnp.max(logits, axis=1, keepdims=True)
    masked = jnp.where(iota == i1, -jnp.inf, logits)
    i2 = jnp.argmax(masked, axis=1, keepdims=True)
    m2 = jnp.max(masked, axis=1, keepdims=True)
    ex = jnp.exp(m2 - m1)
    s = 1.0 + ex
    wgt = (jnp.where(iota == i1, 1.0 / s, 0.0)
           + jnp.where(iota == i2, ex / s, 0.0))

    x = x_ref[...].astype(jnp.bfloat16)
    acc = None
    for e in range(num_experts):
        y = jax.lax.dot_general(
            x, w_ref[e], (((1,), (1,)), ((), ())),
            preferred_element_type=jnp.float32)
        term = wgt[:, e][:, None] * y
        acc = term if acc is None else acc + term
    o_ref[...] = acc


def kernel(inputs, gate_w, expert_w):
    B, T, C = inputs.shape
    E = gate_w.shape[0]
    N = B * T
    x = inputs.reshape(N, C)

    gate_logits = x @ gate_w.T

    w_bf = expert_w.astype(jnp.bfloat16)

    tm = _TM if N % _TM == 0 else N
    out = pl.pallas_call(
        _moe_dense_body,
        out_shape=jax.ShapeDtypeStruct((N, C), jnp.float32),
        grid=(N // tm,),
        in_specs=[
            pl.BlockSpec((tm, C), lambda t: (t, 0)),
            pl.BlockSpec((tm, E), lambda t: (t, 0)),
            pl.BlockSpec((E, C, C), lambda t: (0, 0, 0)),
        ],
        out_specs=pl.BlockSpec((tm, C), lambda t: (t, 0)),
        compiler_params=pltpu.CompilerParams(
            dimension_semantics=("parallel",),
            vmem_limit_bytes=60 * 1024 * 1024,
        ),
    )(x, gate_logits, w_bf)

    return out.astype(inputs.dtype).reshape(B, T, C)


# speedup vs baseline: 1.1106x; 1.1106x over previous
import jax
import jax.numpy as jnp
from jax.experimental import pallas as pl
from jax.experimental.pallas import tpu as pltpu

_TM = 256


def _moe_dense_body(x_ref, lg_ref, w_hbm, o_ref, wbf_ref, stg_ref, sems):
    t = pl.program_id(0)
    num_experts = w_hbm.shape[0]

    @pl.when(t == 0)
    def _load_weights():
        copies = [
            pltpu.make_async_copy(w_hbm.at[e], stg_ref.at[e % 2], sems.at[e % 2])
            for e in range(num_experts)
        ]
        copies[0].start()
        for e in range(num_experts):
            if e + 1 < num_experts:
                copies[e + 1].start()
            copies[e].wait()
            wbf_ref[e] = stg_ref[e % 2].astype(jnp.bfloat16)

    logits = lg_ref[...]
    iota = jax.lax.broadcasted_iota(jnp.int32, logits.shape, 1)

    i1 = jnp.argmax(logits, axis=1, keepdims=True)
    m1 = jnp.max(logits, axis=1, keepdims=True)
    masked = jnp.where(iota == i1, -jnp.inf, logits)
    i2 = jnp.argmax(masked, axis=1, keepdims=True)
    m2 = jnp.max(masked, axis=1, keepdims=True)
    ex = jnp.exp(m2 - m1)
    s = 1.0 + ex
    wgt = (jnp.where(iota == i1, 1.0 / s, 0.0)
           + jnp.where(iota == i2, ex / s, 0.0))

    x = x_ref[...].astype(jnp.bfloat16)
    acc = None
    for e in range(num_experts):
        y = jax.lax.dot_general(
            x, wbf_ref[e], (((1,), (1,)), ((), ())),
            preferred_element_type=jnp.float32)
        term = wgt[:, e][:, None] * y
        acc = term if acc is None else acc + term
    o_ref[...] = acc


def kernel(inputs, gate_w, expert_w):
    B, T, C = inputs.shape
    E = gate_w.shape[0]
    N = B * T
    x = inputs.reshape(N, C)

    gate_logits = x @ gate_w.T

    tm = _TM if N % _TM == 0 else N
    out = pl.pallas_call(
        _moe_dense_body,
        out_shape=jax.ShapeDtypeStruct((N, C), jnp.float32),
        grid=(N // tm,),
        in_specs=[
            pl.BlockSpec((tm, C), lambda t: (t, 0)),
            pl.BlockSpec((tm, E), lambda t: (t, 0)),
            pl.BlockSpec(memory_space=pl.ANY),
        ],
        out_specs=pl.BlockSpec((tm, C), lambda t: (t, 0)),
        scratch_shapes=[
            pltpu.VMEM((E, C, C), jnp.bfloat16),
            pltpu.VMEM((2, C, C), jnp.float32),
            pltpu.SemaphoreType.DMA((2,)),
        ],
        compiler_params=pltpu.CompilerParams(
            dimension_semantics=("arbitrary",),
            vmem_limit_bytes=60 * 1024 * 1024,
        ),
    )(x, gate_logits, expert_w)

    return out.astype(inputs.dtype).reshape(B, T, C)


# speedup vs baseline: 1.1383x; 1.0249x over previous
import jax
import jax.numpy as jnp
from jax.experimental import pallas as pl
from jax.experimental.pallas import tpu as pltpu

_TM = 512


def _moe_dense_body(x_ref, lg_ref, w_hbm, o_ref, wbf_ref, stg_ref, sems):
    t = pl.program_id(0)
    num_experts = w_hbm.shape[0]

    @pl.when(t == 0)
    def _load_weights():
        copies = [
            pltpu.make_async_copy(w_hbm.at[e], stg_ref.at[e % 2], sems.at[e % 2])
            for e in range(num_experts)
        ]
        copies[0].start()
        for e in range(num_experts):
            if e + 1 < num_experts:
                copies[e + 1].start()
            copies[e].wait()
            wbf_ref[e] = stg_ref[e % 2].astype(jnp.bfloat16)

    logits = lg_ref[...]
    iota = jax.lax.broadcasted_iota(jnp.int32, logits.shape, 1)

    i1 = jnp.argmax(logits, axis=1, keepdims=True)
    m1 = jnp.max(logits, axis=1, keepdims=True)
    masked = jnp.where(iota == i1, -jnp.inf, logits)
    i2 = jnp.argmax(masked, axis=1, keepdims=True)
    m2 = jnp.max(masked, axis=1, keepdims=True)
    ex = jnp.exp(m2 - m1)
    s = 1.0 + ex
    wgt = (jnp.where(iota == i1, 1.0 / s, 0.0)
           + jnp.where(iota == i2, ex / s, 0.0))

    x = x_ref[...].astype(jnp.bfloat16)
    acc = None
    for e in range(num_experts):
        y = jax.lax.dot_general(
            x, wbf_ref[e], (((1,), (1,)), ((), ())),
            preferred_element_type=jnp.float32)
        term = wgt[:, e][:, None] * y
        acc = term if acc is None else acc + term
    o_ref[...] = acc


def kernel(inputs, gate_w, expert_w):
    B, T, C = inputs.shape
    E = gate_w.shape[0]
    N = B * T
    x = inputs.reshape(N, C)

    gate_logits = x @ gate_w.T

    tm = _TM if N % _TM == 0 else N
    out = pl.pallas_call(
        _moe_dense_body,
        out_shape=jax.ShapeDtypeStruct((N, C), jnp.float32),
        grid=(N // tm,),
        in_specs=[
            pl.BlockSpec((tm, C), lambda t: (t, 0)),
            pl.BlockSpec((tm, E), lambda t: (t, 0)),
            pl.BlockSpec(memory_space=pl.ANY),
        ],
        out_specs=pl.BlockSpec((tm, C), lambda t: (t, 0)),
        scratch_shapes=[
            pltpu.VMEM((E, C, C), jnp.bfloat16),
            pltpu.VMEM((2, C, C), jnp.float32),
            pltpu.SemaphoreType.DMA((2,)),
        ],
        compiler_params=pltpu.CompilerParams(
            dimension_semantics=("arbitrary",),
            vmem_limit_bytes=60 * 1024 * 1024,
        ),
    )(x, gate_logits, expert_w)

    return out.astype(inputs.dtype).reshape(B, T, C)


# speedup vs baseline: 1.1394x; 1.0010x over previous
import jax
import jax.numpy as jnp
from jax.experimental import pallas as pl
from jax.experimental.pallas import tpu as pltpu

_TM = 1024


def _moe_dense_body(x_ref, lg_ref, w_hbm, o_ref, wbf_ref, stg_ref, sems):
    t = pl.program_id(0)
    num_experts = w_hbm.shape[0]

    @pl.when(t == 0)
    def _load_weights():
        copies = [
            pltpu.make_async_copy(w_hbm.at[e], stg_ref.at[e % 2], sems.at[e % 2])
            for e in range(num_experts)
        ]
        copies[0].start()
        for e in range(num_experts):
            if e + 1 < num_experts:
                copies[e + 1].start()
            copies[e].wait()
            wbf_ref[e] = stg_ref[e % 2].astype(jnp.bfloat16)

    logits = lg_ref[...]
    iota = jax.lax.broadcasted_iota(jnp.int32, logits.shape, 1)

    i1 = jnp.argmax(logits, axis=1, keepdims=True)
    m1 = jnp.max(logits, axis=1, keepdims=True)
    masked = jnp.where(iota == i1, -jnp.inf, logits)
    i2 = jnp.argmax(masked, axis=1, keepdims=True)
    m2 = jnp.max(masked, axis=1, keepdims=True)
    ex = jnp.exp(m2 - m1)
    s = 1.0 + ex
    wgt = (jnp.where(iota == i1, 1.0 / s, 0.0)
           + jnp.where(iota == i2, ex / s, 0.0))

    x = x_ref[...].astype(jnp.bfloat16)
    acc = None
    for e in range(num_experts):
        y = jax.lax.dot_general(
            x, wbf_ref[e], (((1,), (1,)), ((), ())),
            preferred_element_type=jnp.float32)
        term = wgt[:, e][:, None] * y
        acc = term if acc is None else acc + term
    o_ref[...] = acc


def kernel(inputs, gate_w, expert_w):
    B, T, C = inputs.shape
    E = gate_w.shape[0]
    N = B * T
    x = inputs.reshape(N, C)

    gate_logits = x @ gate_w.T

    tm = _TM if N % _TM == 0 else N
    out = pl.pallas_call(
        _moe_dense_body,
        out_shape=jax.ShapeDtypeStruct((N, C), jnp.float32),
        grid=(N // tm,),
        in_specs=[
            pl.BlockSpec((tm, C), lambda t: (t, 0)),
            pl.BlockSpec((tm, E), lambda t: (t, 0)),
            pl.BlockSpec(memory_space=pl.ANY),
        ],
        out_specs=pl.BlockSpec((tm, C), lambda t: (t, 0)),
        scratch_shapes=[
            pltpu.VMEM((E, C, C), jnp.bfloat16),
            pltpu.VMEM((2, C, C), jnp.float32),
            pltpu.SemaphoreType.DMA((2,)),
        ],
        compiler_params=pltpu.CompilerParams(
            dimension_semantics=("arbitrary",),
            vmem_limit_bytes=60 * 1024 * 1024,
        ),
    )(x, gate_logits, expert_w)

    return out.astype(inputs.dtype).reshape(B, T, C)


# speedup vs baseline: 1.1866x; 1.0414x over previous
import jax
import jax.numpy as jnp
from jax.experimental import pallas as pl
from jax.experimental.pallas import tpu as pltpu

_TM = 1024


def _moe_dense_body(x_ref, lg_ref, w_hbm, o_ref, wbf_ref, stg_ref, sems):
    t = pl.program_id(0)
    num_experts = w_hbm.shape[0]

    logits = lg_ref[...]
    iota = jax.lax.broadcasted_iota(jnp.int32, logits.shape, 1)

    i1 = jnp.argmax(logits, axis=1, keepdims=True)
    m1 = jnp.max(logits, axis=1, keepdims=True)
    masked = jnp.where(iota == i1, -jnp.inf, logits)
    i2 = jnp.argmax(masked, axis=1, keepdims=True)
    m2 = jnp.max(masked, axis=1, keepdims=True)
    ex = jnp.exp(m2 - m1)
    s = 1.0 + ex
    wgt = (jnp.where(iota == i1, 1.0 / s, 0.0)
           + jnp.where(iota == i2, ex / s, 0.0))

    x = x_ref[...].astype(jnp.bfloat16)

    def dot_e(e):
        y = jax.lax.dot_general(
            x, wbf_ref[e], (((1,), (1,)), ((), ())),
            preferred_element_type=jnp.float32)
        return wgt[:, e][:, None] * y

    @pl.when(t == 0)
    def _first_step():
        copies = [
            pltpu.make_async_copy(w_hbm.at[e], stg_ref.at[e % 2], sems.at[e % 2])
            for e in range(num_experts)
        ]
        copies[0].start()
        acc = None
        for e in range(num_experts):
            if e + 1 < num_experts:
                copies[e + 1].start()
            copies[e].wait()
            wbf_ref[e] = stg_ref[e % 2].astype(jnp.bfloat16)
            term = dot_e(e)
            acc = term if acc is None else acc + term
        o_ref[...] = acc

    @pl.when(t != 0)
    def _steady_step():
        acc = None
        for e in range(num_experts):
            term = dot_e(e)
            acc = term if acc is None else acc + term
        o_ref[...] = acc


def kernel(inputs, gate_w, expert_w):
    B, T, C = inputs.shape
    E = gate_w.shape[0]
    N = B * T
    x = inputs.reshape(N, C)

    gate_logits = x @ gate_w.T

    tm = _TM if N % _TM == 0 else N
    out = pl.pallas_call(
        _moe_dense_body,
        out_shape=jax.ShapeDtypeStruct((N, C), jnp.float32),
        grid=(N // tm,),
        in_specs=[
            pl.BlockSpec((tm, C), lambda t: (t, 0)),
            pl.BlockSpec((tm, E), lambda t: (t, 0)),
            pl.BlockSpec(memory_space=pl.ANY),
        ],
        out_specs=pl.BlockSpec((tm, C), lambda t: (t, 0)),
        scratch_shapes=[
            pltpu.VMEM((E, C, C), jnp.bfloat16),
            pltpu.VMEM((2, C, C), jnp.float32),
            pltpu.SemaphoreType.DMA((2,)),
        ],
        compiler_params=pltpu.CompilerParams(
            dimension_semantics=("arbitrary",),
            vmem_limit_bytes=60 * 1024 * 1024,
        ),
    )(x, gate_logits, expert_w)

    return out.astype(inputs.dtype).reshape(B, T, C)
